# baseline (speedup 1.0000x reference)
import functools

import jax
import jax.numpy as jnp
from jax import lax
from jax.experimental import pallas as pl
from jax.experimental.pallas import tpu as pltpu
from jax.experimental.pallas import tpu_sc as plsc

_N = 10000
_E = 320000
_D = 128
_NCLS = 4
_NC = 2
_NS = 16
_NW = _NC * _NS
_EPW = _E // _NW
_CHUNK = 80
_EPW_P = 10000
_NCHUNKS = _EPW_P // _CHUNK
_EPAD = _EPW_P - _EPW
_NPAD = 10240
_RPT = _NPAD // _NS

_ROW_BLK = 1000
_NBLK = _N // _ROW_BLK


def _segment_sum_sc(feats, src, dst, zeros):
    mesh = plsc.VectorSubcoreMesh(core_axis_name="c", subcore_axis_name="s")

    @functools.partial(
        pl.kernel,
        out_type=jax.ShapeDtypeStruct((_NC, _NPAD, _D), jnp.float32),
        mesh=mesh,
        scratch_types=[
            pltpu.VMEM_SHARED((_NPAD, _D), jnp.float32),
            pltpu.VMEM((_EPW_P,), jnp.int32),
            pltpu.VMEM((_CHUNK,), jnp.int32),
            pltpu.VMEM((_CHUNK,), jnp.int32),
            pltpu.VMEM((_CHUNK,), jnp.int32),
            pltpu.VMEM((_CHUNK, _D), jnp.float32),
            pltpu.VMEM((_CHUNK, _D), jnp.float32),
            pltpu.VMEM((_CHUNK, _D), jnp.float32),
            pltpu.SemaphoreType.DMA,
            pltpu.SemaphoreType.DMA,
            pltpu.SemaphoreType.DMA,
            pltpu.SemaphoreType.DMA,
            pltpu.SemaphoreType.DMA,
            pltpu.SemaphoreType.DMA,
        ],
    )
    def seg_sum(feats_hbm, src_hbm, dst_hbm, z_hbm, out_hbm,
                acc_sh, src_v, db_a, db_b, db_c, rows_a, rows_b, rows_c,
                sem_a, sem_b, sem_c, semd_a, semd_b, semd_c):
        c = lax.axis_index("c")
        s = lax.axis_index("s")
        wid = s * _NC + c
        r0 = s * _RPT
        base = wid * _EPW_P
        pltpu.sync_copy(src_hbm.at[pl.ds(base, _EPW_P)], src_v)
        pltpu.sync_copy(z_hbm.at[pl.ds(r0, _RPT)], acc_sh.at[pl.ds(r0, _RPT)])
        plsc.subcore_barrier()

        def fire(i, buf, sem):
            pltpu.async_copy(
                feats_hbm.at[src_v.at[pl.ds(i * _CHUNK, _CHUNK)]], buf, sem)

        def fire_dst(i, db, semd):
            pltpu.async_copy(
                dst_hbm.at[pl.ds(base + i * _CHUNK, _CHUNK)], db, semd)

        def wait(i, buf, sem):
            pltpu.make_async_copy(
                feats_hbm.at[src_v.at[pl.ds(i * _CHUNK, _CHUNK)]], buf,
                sem).wait()

        def wait_dst(i, db, semd):
            pltpu.make_async_copy(
                dst_hbm.at[pl.ds(base + i * _CHUNK, _CHUNK)], db, semd).wait()

        rows = (rows_a, rows_b, rows_c)
        sems = (sem_a, sem_b, sem_c)
        dbs = (db_a, db_b, db_c)
        semds = (semd_a, semd_b, semd_c)

        def slot(i, p_wait, p_fire, do_fire):
            if do_fire:
                fire(i + 2, rows[p_fire], sems[p_fire])
                fire_dst(i + 2, dbs[p_fire], semds[p_fire])
            wait(i, rows[p_wait], sems[p_wait])
            wait_dst(i, dbs[p_wait], semds[p_wait])
            pltpu.sync_copy(rows[p_wait], acc_sh.at[dbs[p_wait]], add=True)

        fire(0, rows_a, sem_a)
        fire_dst(0, db_a, semd_a)
        fire(1, rows_b, sem_b)
        fire_dst(1, db_b, semd_b)

        def step(j, carry):
            i0 = 3 * j
            slot(i0, 0, 2, True)
            slot(i0 + 1, 1, 0, True)
            slot(i0 + 2, 2, 1, True)
            return carry

        lax.fori_loop(0, _NCHUNKS // 3, step, 0)
        slot(_NCHUNKS - 2, (_NCHUNKS - 2) % 3, 0, False)
        slot(_NCHUNKS - 1, (_NCHUNKS - 1) % 3, 0, False)

        plsc.subcore_barrier()
        pltpu.sync_copy(acc_sh.at[pl.ds(r0, _RPT)],
                        out_hbm.at[c, pl.ds(r0, _RPT)])

    return seg_sum(feats, src, dst, zeros)


def _layer1_tc(agg, x, W1, R1, b1):
    def body(aA, aB, xr, Wr, Rr, br, hr):
        a = aA[0] + aB[0]
        h = (jnp.dot(a, Wr[...], preferred_element_type=jnp.float32)
             + jnp.dot(xr[...], Rr[...], preferred_element_type=jnp.float32)
             + br[...])
        hr[...] = jnp.maximum(h, 0.0)

    blk = pl.BlockSpec((1, _ROW_BLK, _D), lambda i: (0, i, 0))
    blk2 = pl.BlockSpec((1, _ROW_BLK, _D), lambda i: (1, i, 0))
    rblk = pl.BlockSpec((_ROW_BLK, _D), lambda i: (i, 0))
    wblk = pl.BlockSpec((_D, _D), lambda i: (0, 0))
    bblk = pl.BlockSpec((1, _D), lambda i: (0, 0))
    return pl.pallas_call(
        body,
        grid=(_NBLK,),
        in_specs=[blk, blk2, rblk, wblk, wblk, bblk],
        out_specs=rblk,
        out_shape=jax.ShapeDtypeStruct((_N, _D), jnp.float32),
    )(agg, agg, x, W1, R1, b1.reshape(1, _D))


def _layer2_tc(agg, h1, W2, R2, b2, Wc1, bc1, Wc2p, bc2p):
    def body(aA, aB, h1r, W2r, R2r, b2r, Wc1r, bc1r, Wc2r, bc2r, h2r, outr):
        a = aA[0] + aB[0]
        h2 = (jnp.dot(a, W2r[...], preferred_element_type=jnp.float32)
              + jnp.dot(h1r[...], R2r[...], preferred_element_type=jnp.float32)
              + b2r[...])
        h2r[...] = h2
        t = jnp.maximum(
            jnp.dot(h2, Wc1r[...], preferred_element_type=jnp.float32)
            + bc1r[...], 0.0)
        outr[...] = (jnp.dot(t, Wc2r[...], preferred_element_type=jnp.float32)
                     + bc2r[...])

    blk = pl.BlockSpec((1, _ROW_BLK, _D), lambda i: (0, i, 0))
    blk2 = pl.BlockSpec((1, _ROW_BLK, _D), lambda i: (1, i, 0))
    rblk = pl.BlockSpec((_ROW_BLK, _D), lambda i: (i, 0))
    wblk = pl.BlockSpec((_D, _D), lambda i: (0, 0))
    bblk = pl.BlockSpec((1, _D), lambda i: (0, 0))
    return pl.pallas_call(
        body,
        grid=(_NBLK,),
        in_specs=[blk, blk2, rblk, wblk, wblk, bblk, wblk, bblk, wblk, bblk],
        out_specs=[rblk, rblk],
        out_shape=[jax.ShapeDtypeStruct((_N, _D), jnp.float32),
                   jax.ShapeDtypeStruct((_N, _D), jnp.float32)],
    )(agg, agg, h1, W2, R2, b2.reshape(1, _D), Wc1, bc1.reshape(1, _D),
      Wc2p, bc2p.reshape(1, _D))


def kernel(x, edge_index, W1, b1, R1, W2, b2, R2, Wc1, bc1, Wc2, bc2):
    if _EPAD:
        pad_src = jnp.zeros((_NW, _EPAD), jnp.int32)
        pad_dst = jnp.broadcast_to(_N + jnp.arange(_EPAD, dtype=jnp.int32),
                                   (_NW, _EPAD))
        src = jnp.concatenate(
            [edge_index[0].reshape(_NW, _EPW), pad_src], axis=1).reshape(-1)
        dst = jnp.concatenate(
            [edge_index[1].reshape(_NW, _EPW), pad_dst], axis=1).reshape(-1)
    else:
        src = edge_index[0]
        dst = edge_index[1]
    zeros = jnp.zeros((_NPAD, _D), jnp.float32)

    agg1 = _segment_sum_sc(x, src, dst, zeros)
    h1 = _layer1_tc(agg1, x, W1, R1, b1)
    agg2 = _segment_sum_sc(h1, src, dst, zeros)

    Wc2p = jnp.zeros((_D, _D), jnp.float32).at[:, :_NCLS].set(Wc2)
    bc2p = jnp.zeros((_D,), jnp.float32).at[:_NCLS].set(bc2)
    h2, out_pad = _layer2_tc(agg2, h1, W2, R2, b2, Wc1, bc1, Wc2p, bc2p)
    out = out_pad[:, :_NCLS]

    node_mask = jax.random.uniform(jax.random.key(1), (_N, 1)) > 0.2
    return (out, node_mask, h2)


# speedup vs baseline: 1.0201x; 1.0201x over previous
import functools

import jax
import jax.numpy as jnp
from jax import lax
from jax.experimental import pallas as pl
from jax.experimental.pallas import tpu as pltpu
from jax.experimental.pallas import tpu_sc as plsc

_N = 10000
_E = 320000
_D = 128
_NCLS = 4
_NC = 2
_NS = 16
_NW = _NC * _NS
_EPW = _E // _NW
_CHUNK = 80
_EPW_P = 10000
_NCHUNKS = _EPW_P // _CHUNK
_EPAD = _EPW_P - _EPW
_NPAD = 10240
_RPT = _NPAD // _NS

_ROW_BLK = 1000
_NBLK = _N // _ROW_BLK


def _segment_sum_sc(feats, src, dst, zeros):
    mesh = plsc.VectorSubcoreMesh(core_axis_name="c", subcore_axis_name="s")

    @functools.partial(
        pl.kernel,
        out_type=jax.ShapeDtypeStruct((_NC, _NPAD, _D), jnp.float32),
        mesh=mesh,
        scratch_types=[
            pltpu.VMEM_SHARED((_NPAD, _D), jnp.float32),
            pltpu.VMEM((_EPW_P,), jnp.int32),
            pltpu.VMEM((_CHUNK,), jnp.int32),
            pltpu.VMEM((_CHUNK,), jnp.int32),
            pltpu.VMEM((_CHUNK,), jnp.int32),
            pltpu.VMEM((_CHUNK, _D), jnp.float32),
            pltpu.VMEM((_CHUNK, _D), jnp.float32),
            pltpu.VMEM((_CHUNK, _D), jnp.float32),
            pltpu.SemaphoreType.DMA,
            pltpu.SemaphoreType.DMA,
            pltpu.SemaphoreType.DMA,
            pltpu.SemaphoreType.DMA,
            pltpu.SemaphoreType.DMA,
            pltpu.SemaphoreType.DMA,
            pltpu.SemaphoreType.DMA,
            pltpu.SemaphoreType.DMA,
        ],
    )
    def seg_sum(feats_hbm, src_hbm, dst_hbm, z_hbm, out_hbm,
                acc_sh, src_v, db_a, db_b, db_c, rows_a, rows_b, rows_c,
                sem_a, sem_b, sem_c, semd_a, semd_b, semd_c, sem_s, sem_z):
        c = lax.axis_index("c")
        s = lax.axis_index("s")
        wid = s * _NC + c
        r0 = s * _RPT
        base = wid * _EPW_P
        pltpu.async_copy(src_hbm.at[pl.ds(base, _EPW_P)], src_v, sem_s)
        pltpu.async_copy(z_hbm.at[pl.ds(r0, _RPT)],
                         acc_sh.at[pl.ds(r0, _RPT)], sem_z)
        pltpu.make_async_copy(src_hbm.at[pl.ds(base, _EPW_P)], src_v,
                              sem_s).wait()

        def fire(i, buf, sem):
            pltpu.async_copy(
                feats_hbm.at[src_v.at[pl.ds(i * _CHUNK, _CHUNK)]], buf, sem)

        def fire_dst(i, db, semd):
            pltpu.async_copy(
                dst_hbm.at[pl.ds(base + i * _CHUNK, _CHUNK)], db, semd)

        def wait(i, buf, sem):
            pltpu.make_async_copy(
                feats_hbm.at[src_v.at[pl.ds(i * _CHUNK, _CHUNK)]], buf,
                sem).wait()

        def wait_dst(i, db, semd):
            pltpu.make_async_copy(
                dst_hbm.at[pl.ds(base + i * _CHUNK, _CHUNK)], db, semd).wait()

        rows = (rows_a, rows_b, rows_c)
        sems = (sem_a, sem_b, sem_c)
        dbs = (db_a, db_b, db_c)
        semds = (semd_a, semd_b, semd_c)

        def slot(i, p_wait, p_fire, do_fire):
            if do_fire:
                fire(i + 2, rows[p_fire], sems[p_fire])
                fire_dst(i + 2, dbs[p_fire], semds[p_fire])
            wait(i, rows[p_wait], sems[p_wait])
            wait_dst(i, dbs[p_wait], semds[p_wait])
            pltpu.sync_copy(rows[p_wait], acc_sh.at[dbs[p_wait]], add=True)

        fire(0, rows_a, sem_a)
        fire_dst(0, db_a, semd_a)
        fire(1, rows_b, sem_b)
        fire_dst(1, db_b, semd_b)
        pltpu.make_async_copy(z_hbm.at[pl.ds(r0, _RPT)],
                              acc_sh.at[pl.ds(r0, _RPT)], sem_z).wait()
        plsc.subcore_barrier()

        def step(j, carry):
            i0 = 3 * j
            slot(i0, 0, 2, True)
            slot(i0 + 1, 1, 0, True)
            slot(i0 + 2, 2, 1, True)
            return carry

        lax.fori_loop(0, _NCHUNKS // 3, step, 0)
        slot(_NCHUNKS - 2, (_NCHUNKS - 2) % 3, 0, False)
        slot(_NCHUNKS - 1, (_NCHUNKS - 1) % 3, 0, False)

        plsc.subcore_barrier()
        pltpu.sync_copy(acc_sh.at[pl.ds(r0, _RPT)],
                        out_hbm.at[c, pl.ds(r0, _RPT)])

    return seg_sum(feats, src, dst, zeros)


def _layer1_tc(agg, x, W1, R1, b1):
    def body(aA, aB, xr, Wr, Rr, br, hr):
        a = aA[0] + aB[0]
        h = (jnp.dot(a, Wr[...], preferred_element_type=jnp.float32)
             + jnp.dot(xr[...], Rr[...], preferred_element_type=jnp.float32)
             + br[...])
        hr[...] = jnp.maximum(h, 0.0)

    blk = pl.BlockSpec((1, _ROW_BLK, _D), lambda i: (0, i, 0))
    blk2 = pl.BlockSpec((1, _ROW_BLK, _D), lambda i: (1, i, 0))
    rblk = pl.BlockSpec((_ROW_BLK, _D), lambda i: (i, 0))
    wblk = pl.BlockSpec((_D, _D), lambda i: (0, 0))
    bblk = pl.BlockSpec((1, _D), lambda i: (0, 0))
    return pl.pallas_call(
        body,
        grid=(_NBLK,),
        in_specs=[blk, blk2, rblk, wblk, wblk, bblk],
        out_specs=rblk,
        out_shape=jax.ShapeDtypeStruct((_N, _D), jnp.float32),
    )(agg, agg, x, W1, R1, b1.reshape(1, _D))


def _layer2_tc(agg, h1, W2, R2, b2, Wc1, bc1, Wc2p, bc2p):
    def body(aA, aB, h1r, W2r, R2r, b2r, Wc1r, bc1r, Wc2r, bc2r, h2r, outr):
        a = aA[0] + aB[0]
        h2 = (jnp.dot(a, W2r[...], preferred_element_type=jnp.float32)
              + jnp.dot(h1r[...], R2r[...], preferred_element_type=jnp.float32)
              + b2r[...])
        h2r[...] = h2
        t = jnp.maximum(
            jnp.dot(h2, Wc1r[...], preferred_element_type=jnp.float32)
            + bc1r[...], 0.0)
        outr[...] = (jnp.dot(t, Wc2r[...], preferred_element_type=jnp.float32)
                     + bc2r[...])

    blk = pl.BlockSpec((1, _ROW_BLK, _D), lambda i: (0, i, 0))
    blk2 = pl.BlockSpec((1, _ROW_BLK, _D), lambda i: (1, i, 0))
    rblk = pl.BlockSpec((_ROW_BLK, _D), lambda i: (i, 0))
    wblk = pl.BlockSpec((_D, _D), lambda i: (0, 0))
    bblk = pl.BlockSpec((1, _D), lambda i: (0, 0))
    return pl.pallas_call(
        body,
        grid=(_NBLK,),
        in_specs=[blk, blk2, rblk, wblk, wblk, bblk, wblk, bblk, wblk, bblk],
        out_specs=[rblk, rblk],
        out_shape=[jax.ShapeDtypeStruct((_N, _D), jnp.float32),
                   jax.ShapeDtypeStruct((_N, _D), jnp.float32)],
    )(agg, agg, h1, W2, R2, b2.reshape(1, _D), Wc1, bc1.reshape(1, _D),
      Wc2p, bc2p.reshape(1, _D))


def kernel(x, edge_index, W1, b1, R1, W2, b2, R2, Wc1, bc1, Wc2, bc2):
    if _EPAD:
        pad_src = jnp.zeros((_NW, _EPAD), jnp.int32)
        pad_dst = jnp.broadcast_to(_N + jnp.arange(_EPAD, dtype=jnp.int32),
                                   (_NW, _EPAD))
        src = jnp.concatenate(
            [edge_index[0].reshape(_NW, _EPW), pad_src], axis=1).reshape(-1)
        dst = jnp.concatenate(
            [edge_index[1].reshape(_NW, _EPW), pad_dst], axis=1).reshape(-1)
    else:
        src = edge_index[0]
        dst = edge_index[1]
    zeros = jnp.zeros((_NPAD, _D), jnp.float32)

    agg1 = _segment_sum_sc(x, src, dst, zeros)
    h1 = _layer1_tc(agg1, x, W1, R1, b1)
    agg2 = _segment_sum_sc(h1, src, dst, zeros)

    Wc2p = jnp.zeros((_D, _D), jnp.float32).at[:, :_NCLS].set(Wc2)
    bc2p = jnp.zeros((_D,), jnp.float32).at[:_NCLS].set(bc2)
    h2, out_pad = _layer2_tc(agg2, h1, W2, R2, b2, Wc1, bc1, Wc2p, bc2p)
    out = out_pad[:, :_NCLS]

    node_mask = jax.random.uniform(jax.random.key(1), (_N, 1)) > 0.2
    return (out, node_mask, h2)


# speedup vs baseline: 1.0206x; 1.0004x over previous
import functools

import jax
import jax.numpy as jnp
from jax import lax
from jax.experimental import pallas as pl
from jax.experimental.pallas import tpu as pltpu
from jax.experimental.pallas import tpu_sc as plsc

_N = 10000
_E = 320000
_D = 128
_NCLS = 4
_NC = 2
_NS = 16
_NW = _NC * _NS
_EPW = _E // _NW
_CHUNK = 80
_EPW_P = 10000
_NCHUNKS = _EPW_P // _CHUNK
_EPAD = _EPW_P - _EPW
_NPAD = 10240
_RPT = _NPAD // _NS

_ROW_BLK = 1000
_NBLK = _N // _ROW_BLK


def _segment_sum_sc(feats, src, dst, zeros):
    mesh = plsc.VectorSubcoreMesh(core_axis_name="c", subcore_axis_name="s")

    @functools.partial(
        pl.kernel,
        out_type=jax.ShapeDtypeStruct((_NC, _NPAD, _D), jnp.float32),
        mesh=mesh,
        scratch_types=[
            pltpu.VMEM_SHARED((_NPAD, _D), jnp.float32),
            pltpu.VMEM((_EPW_P,), jnp.int32),
            pltpu.VMEM((_CHUNK,), jnp.int32),
            pltpu.VMEM((_CHUNK,), jnp.int32),
            pltpu.VMEM((_CHUNK,), jnp.int32),
            pltpu.VMEM((_CHUNK, _D), jnp.float32),
            pltpu.VMEM((_CHUNK, _D), jnp.float32),
            pltpu.VMEM((_CHUNK, _D), jnp.float32),
            pltpu.SemaphoreType.DMA,
            pltpu.SemaphoreType.DMA,
            pltpu.SemaphoreType.DMA,
            pltpu.SemaphoreType.DMA,
            pltpu.SemaphoreType.DMA,
            pltpu.SemaphoreType.DMA,
            pltpu.SemaphoreType.DMA,
            pltpu.SemaphoreType.DMA,
        ],
    )
    def seg_sum(feats_hbm, src_hbm, dst_hbm, z_hbm, out_hbm,
                acc_sh, src_v, db_a, db_b, db_c, rows_a, rows_b, rows_c,
                sem_a, sem_b, sem_c, semd_a, semd_b, semd_c, sem_s, sem_z):
        c = lax.axis_index("c")
        s = lax.axis_index("s")
        wid = s * _NC + c
        r0 = s * _RPT
        base = wid * _EPW_P
        pltpu.async_copy(src_hbm.at[pl.ds(base, _EPW_P)], src_v, sem_s)
        pltpu.async_copy(z_hbm.at[pl.ds(r0, _RPT)],
                         acc_sh.at[pl.ds(r0, _RPT)], sem_z)
        pltpu.make_async_copy(src_hbm.at[pl.ds(base, _EPW_P)], src_v,
                              sem_s).wait()

        def fire(i, buf, sem):
            pltpu.async_copy(
                feats_hbm.at[src_v.at[pl.ds(i * _CHUNK, _CHUNK)]], buf, sem)

        def fire_dst(i, db, semd):
            pltpu.async_copy(
                dst_hbm.at[pl.ds(base + i * _CHUNK, _CHUNK)], db, semd)

        def wait(i, buf, sem):
            pltpu.make_async_copy(
                feats_hbm.at[src_v.at[pl.ds(i * _CHUNK, _CHUNK)]], buf,
                sem).wait()

        def wait_dst(i, db, semd):
            pltpu.make_async_copy(
                dst_hbm.at[pl.ds(base + i * _CHUNK, _CHUNK)], db, semd).wait()

        rows = (rows_a, rows_b, rows_c)
        sems = (sem_a, sem_b, sem_c)
        dbs = (db_a, db_b, db_c)
        semds = (semd_a, semd_b, semd_c)

        def slot(i, p_wait, p_fire, do_fire):
            if do_fire:
                fire(i + 2, rows[p_fire], sems[p_fire])
                fire_dst(i + 2, dbs[p_fire], semds[p_fire])
            wait(i, rows[p_wait], sems[p_wait])
            wait_dst(i, dbs[p_wait], semds[p_wait])
            pltpu.sync_copy(rows[p_wait], acc_sh.at[dbs[p_wait]], add=True)

        fire(0, rows_a, sem_a)
        fire_dst(0, db_a, semd_a)
        fire(1, rows_b, sem_b)
        fire_dst(1, db_b, semd_b)
        pltpu.make_async_copy(z_hbm.at[pl.ds(r0, _RPT)],
                              acc_sh.at[pl.ds(r0, _RPT)], sem_z).wait()
        plsc.subcore_barrier()

        def step(j, carry):
            i0 = 3 * j
            slot(i0, 0, 2, True)
            slot(i0 + 1, 1, 0, True)
            slot(i0 + 2, 2, 1, True)
            return carry

        lax.fori_loop(0, _NCHUNKS // 3, step, 0)
        slot(_NCHUNKS - 2, (_NCHUNKS - 2) % 3, 0, False)
        slot(_NCHUNKS - 1, (_NCHUNKS - 1) % 3, 0, False)

        plsc.subcore_barrier()
        pltpu.sync_copy(acc_sh.at[pl.ds(r0, _RPT)],
                        out_hbm.at[c, pl.ds(r0, _RPT)])

    return seg_sum(feats, src, dst, zeros)


def _layer1_tc(agg, x, W1, R1, b1):
    def body(aA, aB, xr, Wr, Rr, br, hr):
        a = aA[0] + aB[0]
        h = (jnp.dot(a, Wr[...], preferred_element_type=jnp.float32)
             + jnp.dot(xr[...], Rr[...], preferred_element_type=jnp.float32)
             + br[...])
        hr[...] = jnp.maximum(h, 0.0)

    blk = pl.BlockSpec((1, _ROW_BLK, _D), lambda i: (0, i, 0))
    blk2 = pl.BlockSpec((1, _ROW_BLK, _D), lambda i: (1, i, 0))
    rblk = pl.BlockSpec((_ROW_BLK, _D), lambda i: (i, 0))
    wblk = pl.BlockSpec((_D, _D), lambda i: (0, 0))
    bblk = pl.BlockSpec((1, _D), lambda i: (0, 0))
    return pl.pallas_call(
        body,
        grid=(_NBLK,),
        in_specs=[blk, blk2, rblk, wblk, wblk, bblk],
        out_specs=rblk,
        out_shape=jax.ShapeDtypeStruct((_N, _D), jnp.float32),
    )(agg, agg, x, W1, R1, b1.reshape(1, _D))


def _layer2_tc(agg, h1, W2, R2, b2, Wc1, bc1, Wc2p, bc2p):
    def body(aA, aB, h1r, W2r, R2r, b2r, Wc1r, bc1r, Wc2r, bc2r, h2r, outr):
        a = aA[0] + aB[0]
        h2 = (jnp.dot(a, W2r[...], preferred_element_type=jnp.float32)
              + jnp.dot(h1r[...], R2r[...], preferred_element_type=jnp.float32)
              + b2r[...])
        h2r[...] = h2
        t = jnp.maximum(
            jnp.dot(h2, Wc1r[...], preferred_element_type=jnp.float32)
            + bc1r[...], 0.0)
        o = (jnp.dot(t, Wc2r[...], preferred_element_type=jnp.float32)
             + bc2r[...])
        outr[...] = o[:, :_NCLS]

    blk = pl.BlockSpec((1, _ROW_BLK, _D), lambda i: (0, i, 0))
    blk2 = pl.BlockSpec((1, _ROW_BLK, _D), lambda i: (1, i, 0))
    rblk = pl.BlockSpec((_ROW_BLK, _D), lambda i: (i, 0))
    wblk = pl.BlockSpec((_D, _D), lambda i: (0, 0))
    bblk = pl.BlockSpec((1, _D), lambda i: (0, 0))
    return pl.pallas_call(
        body,
        grid=(_NBLK,),
        in_specs=[blk, blk2, rblk, wblk, wblk, bblk, wblk, bblk, wblk, bblk],
        out_specs=[rblk, pl.BlockSpec((_ROW_BLK, _NCLS), lambda i: (i, 0))],
        out_shape=[jax.ShapeDtypeStruct((_N, _D), jnp.float32),
                   jax.ShapeDtypeStruct((_N, _NCLS), jnp.float32)],
    )(agg, agg, h1, W2, R2, b2.reshape(1, _D), Wc1, bc1.reshape(1, _D),
      Wc2p, bc2p.reshape(1, _D))


def kernel(x, edge_index, W1, b1, R1, W2, b2, R2, Wc1, bc1, Wc2, bc2):
    if _EPAD:
        pad_src = jnp.zeros((_NW, _EPAD), jnp.int32)
        pad_dst = jnp.broadcast_to(_N + jnp.arange(_EPAD, dtype=jnp.int32),
                                   (_NW, _EPAD))
        src = jnp.concatenate(
            [edge_index[0].reshape(_NW, _EPW), pad_src], axis=1).reshape(-1)
        dst = jnp.concatenate(
            [edge_index[1].reshape(_NW, _EPW), pad_dst], axis=1).reshape(-1)
    else:
        src = edge_index[0]
        dst = edge_index[1]
    zeros = jnp.zeros((_NPAD, _D), jnp.float32)

    agg1 = _segment_sum_sc(x, src, dst, zeros)
    h1 = _layer1_tc(agg1, x, W1, R1, b1)
    agg2 = _segment_sum_sc(h1, src, dst, zeros)

    Wc2p = jnp.zeros((_D, _D), jnp.float32).at[:, :_NCLS].set(Wc2)
    bc2p = jnp.zeros((_D,), jnp.float32).at[:_NCLS].set(bc2)
    h2, out = _layer2_tc(agg2, h1, W2, R2, b2, Wc1, bc1, Wc2p, bc2p)

    node_mask = jax.random.uniform(jax.random.key(1), (_N, 1)) > 0.2
    return (out, node_mask, h2)
